# baseline (speedup 1.0000x reference)
import functools

import jax
import jax.numpy as jnp
from jax import lax
from jax.experimental import pallas as pl
from jax.experimental.pallas import tpu as pltpu
from jax.experimental.pallas import tpu_sc as plsc

NW = 32
CHUNK = 128
NBUF = 4
VT = 128


def _make_convert(V, D):
    n_slabs_total = V // VT
    tail = V % VT
    n_slabs = n_slabs_total // NW
    rem = n_slabs_total - n_slabs * NW
    mesh = plsc.VectorSubcoreMesh(core_axis_name="c", subcore_axis_name="s")

    @functools.partial(
        pl.kernel,
        mesh=mesh,
        out_type=jax.ShapeDtypeStruct((V * D,), jnp.float32),
        scratch_types=[
            pltpu.VMEM((2 * D, VT), jnp.float32),
            pltpu.VMEM((2 * VT * D,), jnp.float32),
            pltpu.SemaphoreType.DMA((2,)),
            pltpu.SemaphoreType.DMA((2,)),
        ],
        compiler_params=pltpu.CompilerParams(
            use_tc_tiling_on_sc=True, needs_layout_passes=False),
    )
    def convert_kernel(tt_hbm, tail_hbm, out_hbm, in_v, tr_v, isem, osem):
        wid = lax.axis_index("s") * 2 + lax.axis_index("c")
        my_n = n_slabs + jnp.where(wid < rem, 1, 0)
        base = wid * n_slabs + jnp.minimum(wid, rem)

        lanes_d = lax.iota(jnp.int32, 16) * D

        def v0(s):
            return (base + s) * VT

        def start_in(s, b):
            pltpu.async_copy(
                tt_hbm.at[:, pl.ds(v0(s), VT)],
                in_v.at[pl.ds(b * D, D)], isem.at[b])

        def wait_in(b):
            pltpu.make_async_copy(
                tt_hbm.at[:, pl.ds(0, VT)],
                in_v.at[pl.ds(b * D, D)], isem.at[b]).wait()

        def start_out(s, b):
            pltpu.async_copy(
                tr_v.at[pl.ds(b * VT * D, VT * D)],
                out_hbm.at[pl.ds(v0(s) * D, VT * D)], osem.at[b])

        def wait_out(b):
            pltpu.make_async_copy(
                tr_v.at[pl.ds(b * VT * D, VT * D)],
                out_hbm.at[pl.ds(0, VT * D)], osem.at[b]).wait()

        def transpose(b):
            for d in range(D):
                for k in range(VT // 16):
                    vals = in_v[b * D + d, pl.ds(k * 16, 16)]
                    plsc.store_scatter(
                        tr_v,
                        [lanes_d + (b * VT * D + k * 16 * D + d)], vals)

        if tail:
            @pl.when(wid == NW - 1)
            def _():
                pltpu.sync_copy(tail_hbm, in_v.at[pl.ds(0, D)])
                for d in range(D):
                    for k in range(VT // 16):
                        vals = in_v[d, pl.ds(k * 16, 16)]
                        plsc.store_scatter(
                            tr_v, [lanes_d + (k * 16 * D + d)], vals)
                pltpu.sync_copy(
                    tr_v.at[pl.ds(0, VT * D)],
                    out_hbm.at[pl.ds((V - VT) * D, VT * D)])

        start_in(0, 0)

        def step2(s2, _):
            s0 = s2 * 2

            wait_in(0)

            @pl.when(s0 + 1 < my_n)
            def _():
                start_in(s0 + 1, 1)

            @pl.when(s0 >= 2)
            def _():
                wait_out(0)

            transpose(0)
            start_out(s0, 0)

            @pl.when(s0 + 1 < my_n)
            def _():
                wait_in(1)

                @pl.when(s0 + 2 < my_n)
                def _():
                    start_in(s0 + 2, 0)

                @pl.when(s0 >= 1)
                def _():
                    wait_out(1)

                transpose(1)
                start_out(s0 + 1, 1)

            return 0

        lax.fori_loop(0, (my_n + 1) // 2, step2, 0)

        wait_out(0)

        @pl.when(my_n >= 2)
        def _():
            wait_out(1)

    return convert_kernel


def _make_gather(P, V, D):
    per_w = P // NW
    n_chunks = per_w // CHUNK
    n_groups = n_chunks // NBUF
    assert n_chunks % NBUF == 0 and per_w % CHUNK == 0 and P % NW == 0
    mesh = plsc.VectorSubcoreMesh(core_axis_name="c", subcore_axis_name="s")

    @functools.partial(
        pl.kernel,
        mesh=mesh,
        out_type=jax.ShapeDtypeStruct((P, D), jnp.float32),
        scratch_types=[
            pltpu.VMEM((n_chunks, CHUNK), jnp.int32),
            pltpu.VMEM((NBUF, CHUNK, D), jnp.float32),
            pltpu.SemaphoreType.DMA((NBUF,)),
            pltpu.SemaphoreType.DMA((NBUF,)),
        ],
        compiler_params=pltpu.CompilerParams(use_tc_tiling_on_sc=False),
    )
    def gather_kernel(x_hbm, table_hbm, out_hbm, idx_v, rows_v, gsem, osem):
        wid = lax.axis_index("s") * 2 + lax.axis_index("c")
        base = wid * per_w

        pltpu.sync_copy(x_hbm.at[pl.ds(wid * n_chunks, n_chunks)], idx_v)

        def start_gather(j, b):
            pltpu.async_copy(
                table_hbm.at[idx_v.at[j]], rows_v.at[b], gsem.at[b])

        def wait_gather(j, b):
            pltpu.make_async_copy(
                table_hbm.at[idx_v.at[j]], rows_v.at[b], gsem.at[b]).wait()

        def start_out(j, b):
            pltpu.async_copy(
                rows_v.at[b], out_hbm.at[pl.ds(base + j * CHUNK, CHUNK)],
                osem.at[b])

        def wait_out(j, b):
            pltpu.make_async_copy(
                rows_v.at[b], out_hbm.at[pl.ds(base + j * CHUNK, CHUNK)],
                osem.at[b]).wait()

        for b in range(NBUF):
            start_gather(b, b)
        wait_gather(0, 0)
        start_out(0, 0)

        def group(g, _):
            for b in range(NBUF):
                step = g * NBUF + b
                wait_out(step - NBUF, b)
                start_gather(step, b)
                j_w = step - (NBUF - 1)
                bw = (b + 1) % NBUF
                wait_gather(j_w, bw)
                start_out(j_w, bw)
            return 0

        lax.fori_loop(1, n_groups, group, 0)

        for s in range(NBUF - 1):
            j_w = n_chunks - (NBUF - 1) + s
            wait_gather(j_w, j_w % NBUF)
            start_out(j_w, j_w % NBUF)
        for s in range(NBUF):
            j_w = n_chunks - NBUF + s
            wait_out(j_w, j_w % NBUF)

    return gather_kernel


def kernel(x, table):
    B, L = x.shape
    V, D = table.shape
    P = B * L
    xf = x.reshape(P // CHUNK, CHUNK).astype(jnp.int32)
    tt = table.T
    tlin = _make_convert(V, D)(tt, tt[:, V - VT:])
    out = _make_gather(P, V, D)(xf, tlin.reshape(V, D))
    return out.reshape(B, L, D)


# speedup vs baseline: 1.2393x; 1.2393x over previous
import functools

import jax
import jax.numpy as jnp
from jax import lax
from jax.experimental import pallas as pl
from jax.experimental.pallas import tpu as pltpu
from jax.experimental.pallas import tpu_sc as plsc

NW = 32
CHUNK = 128
NBUF = 4
VT = 128


def _make_convert(V, D):
    n_slabs_total = V // VT
    tail = V % VT
    n_slabs = n_slabs_total // NW
    rem = n_slabs_total - n_slabs * NW
    mesh = plsc.VectorSubcoreMesh(core_axis_name="c", subcore_axis_name="s")

    @functools.partial(
        pl.kernel,
        mesh=mesh,
        out_type=jax.ShapeDtypeStruct((V * D,), jnp.float32),
        scratch_types=[
            pltpu.VMEM((2 * D, VT), jnp.float32),
            pltpu.VMEM((2 * VT * D,), jnp.float32),
            pltpu.SemaphoreType.DMA((2,)),
            pltpu.SemaphoreType.DMA((2,)),
        ],
        compiler_params=pltpu.CompilerParams(
            use_tc_tiling_on_sc=True, needs_layout_passes=False),
    )
    def convert_kernel(tt_hbm, tail_hbm, out_hbm, in_v, tr_v, isem, osem):
        wid = lax.axis_index("s") * 2 + lax.axis_index("c")
        my_n = n_slabs + jnp.where(wid < rem, 1, 0)
        base = wid * n_slabs + jnp.minimum(wid, rem)

        lanes_d = lax.iota(jnp.int32, 16) * D

        def v0(s):
            return (base + s) * VT

        def start_in(s, b):
            pltpu.async_copy(
                tt_hbm.at[:, pl.ds(v0(s), VT)],
                in_v.at[pl.ds(b * D, D)], isem.at[b])

        def wait_in(b):
            pltpu.make_async_copy(
                tt_hbm.at[:, pl.ds(0, VT)],
                in_v.at[pl.ds(b * D, D)], isem.at[b]).wait()

        def start_out(s, b):
            pltpu.async_copy(
                tr_v.at[pl.ds(b * VT * D, VT * D)],
                out_hbm.at[pl.ds(v0(s) * D, VT * D)], osem.at[b])

        def wait_out(b):
            pltpu.make_async_copy(
                tr_v.at[pl.ds(b * VT * D, VT * D)],
                out_hbm.at[pl.ds(0, VT * D)], osem.at[b]).wait()

        def transpose(b):
            @plsc.parallel_loop(0, D, unroll=8)
            def _(d):
                for k in range(VT // 16):
                    vals = in_v[b * D + d, pl.ds(k * 16, 16)]
                    plsc.store_scatter(
                        tr_v,
                        [lanes_d + (b * VT * D + k * 16 * D) + d], vals)

        if tail:
            @pl.when(wid == NW - 1)
            def _():
                pltpu.sync_copy(tail_hbm, in_v.at[pl.ds(0, D)])
                @plsc.parallel_loop(0, D, unroll=8)
                def _(d):
                    for k in range(VT // 16):
                        vals = in_v[d, pl.ds(k * 16, 16)]
                        plsc.store_scatter(
                            tr_v, [lanes_d + (k * 16 * D) + d], vals)
                pltpu.sync_copy(
                    tr_v.at[pl.ds(0, VT * D)],
                    out_hbm.at[pl.ds((V - VT) * D, VT * D)])

        start_in(0, 0)

        def step2(s2, _):
            s0 = s2 * 2

            wait_in(0)

            @pl.when(s0 + 1 < my_n)
            def _():
                start_in(s0 + 1, 1)

            @pl.when(s0 >= 2)
            def _():
                wait_out(0)

            transpose(0)
            start_out(s0, 0)

            @pl.when(s0 + 1 < my_n)
            def _():
                wait_in(1)

                @pl.when(s0 + 2 < my_n)
                def _():
                    start_in(s0 + 2, 0)

                @pl.when(s0 >= 1)
                def _():
                    wait_out(1)

                transpose(1)
                start_out(s0 + 1, 1)

            return 0

        lax.fori_loop(0, (my_n + 1) // 2, step2, 0)

        wait_out(0)

        @pl.when(my_n >= 2)
        def _():
            wait_out(1)

    return convert_kernel


def _make_gather(P, V, D):
    per_w = P // NW
    n_chunks = per_w // CHUNK
    n_groups = n_chunks // NBUF
    assert n_chunks % NBUF == 0 and per_w % CHUNK == 0 and P % NW == 0
    mesh = plsc.VectorSubcoreMesh(core_axis_name="c", subcore_axis_name="s")

    @functools.partial(
        pl.kernel,
        mesh=mesh,
        out_type=jax.ShapeDtypeStruct((P, D), jnp.float32),
        scratch_types=[
            pltpu.VMEM((n_chunks, CHUNK), jnp.int32),
            pltpu.VMEM((NBUF, CHUNK, D), jnp.float32),
            pltpu.SemaphoreType.DMA((NBUF,)),
            pltpu.SemaphoreType.DMA((NBUF,)),
        ],
        compiler_params=pltpu.CompilerParams(use_tc_tiling_on_sc=False),
    )
    def gather_kernel(x_hbm, table_hbm, out_hbm, idx_v, rows_v, gsem, osem):
        wid = lax.axis_index("s") * 2 + lax.axis_index("c")
        base = wid * per_w

        pltpu.sync_copy(x_hbm.at[pl.ds(wid * n_chunks, n_chunks)], idx_v)

        def start_gather(j, b):
            pltpu.async_copy(
                table_hbm.at[idx_v.at[j]], rows_v.at[b], gsem.at[b])

        def wait_gather(j, b):
            pltpu.make_async_copy(
                table_hbm.at[idx_v.at[j]], rows_v.at[b], gsem.at[b]).wait()

        def start_out(j, b):
            pltpu.async_copy(
                rows_v.at[b], out_hbm.at[pl.ds(base + j * CHUNK, CHUNK)],
                osem.at[b])

        def wait_out(j, b):
            pltpu.make_async_copy(
                rows_v.at[b], out_hbm.at[pl.ds(base + j * CHUNK, CHUNK)],
                osem.at[b]).wait()

        for b in range(NBUF):
            start_gather(b, b)
        wait_gather(0, 0)
        start_out(0, 0)

        def group(g, _):
            for b in range(NBUF):
                step = g * NBUF + b
                wait_out(step - NBUF, b)
                start_gather(step, b)
                j_w = step - (NBUF - 1)
                bw = (b + 1) % NBUF
                wait_gather(j_w, bw)
                start_out(j_w, bw)
            return 0

        lax.fori_loop(1, n_groups, group, 0)

        for s in range(NBUF - 1):
            j_w = n_chunks - (NBUF - 1) + s
            wait_gather(j_w, j_w % NBUF)
            start_out(j_w, j_w % NBUF)
        for s in range(NBUF):
            j_w = n_chunks - NBUF + s
            wait_out(j_w, j_w % NBUF)

    return gather_kernel


def kernel(x, table):
    B, L = x.shape
    V, D = table.shape
    P = B * L
    xf = x.reshape(P // CHUNK, CHUNK).astype(jnp.int32)
    tt = table.T
    tlin = _make_convert(V, D)(tt, tt[:, V - VT:])
    out = _make_gather(P, V, D)(xf, tlin.reshape(V, D))
    return out.reshape(B, L, D)
